# baseline (speedup 1.0000x reference)
import functools

import jax
import jax.numpy as jnp
from jax import lax
from jax.experimental import pallas as pl
from jax.experimental.pallas import tpu as pltpu
from jax.experimental.pallas import tpu_sc as plsc

VOCAB = 100000
CTX = 32
EMBED = 7
HIDDEN = 64
BATCH = 1024
EPAD = 16
CHUNK = 128
BT = 16
NB = BATCH // BT


@functools.lru_cache(maxsize=None)
def _make_gather():
    nc, ns = 2, 16
    nw = nc * ns
    total = BATCH * CTX
    rows_per_w = total // nw
    n_chunks = rows_per_w // CHUNK
    mesh = plsc.VectorSubcoreMesh(
        core_axis_name="c", subcore_axis_name="s", num_cores=nc, num_subcores=ns
    )

    @functools.partial(
        pl.kernel,
        mesh=mesh,
        compiler_params=pltpu.CompilerParams(use_tc_tiling_on_sc=False),
        out_type=jax.ShapeDtypeStruct((total, EPAD), jnp.float32),
        scratch_types=[
            pltpu.VMEM((n_chunks, CHUNK), jnp.int32),
            pltpu.VMEM((rows_per_w, EPAD), jnp.float32),
            pltpu.SemaphoreType.DMA,
        ],
    )
    def gather_k(idx_hbm, table_hbm, out_hbm, idx_v, rows_v, sem):
        wid = lax.axis_index("s") * nc + lax.axis_index("c")
        pltpu.sync_copy(idx_hbm.at[pl.ds(wid * n_chunks, n_chunks)], idx_v)
        copies = [
            pltpu.async_copy(
                table_hbm.at[idx_v.at[j]],
                rows_v.at[pl.ds(j * CHUNK, CHUNK)],
                sem,
            )
            for j in range(n_chunks)
        ]
        for c in copies:
            c.wait()
        pltpu.sync_copy(rows_v, out_hbm.at[pl.ds(wid * rows_per_w, rows_per_w)])

    return gather_k


def _fc1_body(e_ref, w1_ref, b1_ref, h_ref):
    h = lax.dot_general(
        e_ref[...], w1_ref[...], (((1,), (1,)), ((), ())),
        preferred_element_type=jnp.float32,
    )
    h_ref[...] = jnp.maximum(h + b1_ref[...], 0.0)


_fc1 = pl.pallas_call(
    _fc1_body,
    out_shape=jax.ShapeDtypeStruct((BATCH, HIDDEN), jnp.float32),
)


RING = 2


def _fc2_body(
    h_ref, w2_hbm, b2_hbm, out_hbm, w2_v, b2_v, r0, r1, sem, sem2, s0, s1
):
    j = pl.program_id(0)
    rings = [r0, r1]
    sems = [s0, s1]
    slot = lax.rem(j, RING)

    @pl.when(j == 0)
    def _():
        w2_cp = pltpu.make_async_copy(w2_hbm, w2_v, sem)
        b2_cp = pltpu.make_async_copy(b2_hbm, b2_v, sem2)
        w2_cp.start()
        b2_cp.start()
        w2_cp.wait()
        b2_cp.wait()

    for s in range(RING):
        @pl.when(slot == s)
        def _(s=s):
            buf, osem = rings[s], sems[s]

            @pl.when(j >= RING)
            def _():
                pltpu.make_async_copy(
                    buf, out_hbm.at[pl.ds((j - RING) * BT, BT), :], osem
                ).wait()

            buf[...] = (
                lax.dot_general(
                    h_ref[...].astype(jnp.bfloat16), w2_v[...],
                    (((1,), (1,)), ((), ())),
                    preferred_element_type=jnp.float32,
                )
                + b2_v[...]
            )
            pltpu.make_async_copy(
                buf, out_hbm.at[pl.ds(j * BT, BT), :], osem
            ).start()

    @pl.when(j == NB - 1)
    def _():
        for k in range(RING):
            step = NB - RING + k
            pltpu.make_async_copy(
                rings[step % RING],
                out_hbm.at[pl.ds(step * BT, BT), :],
                sems[step % RING],
            ).wait()


_fc2 = pl.pallas_call(
    _fc2_body,
    grid=(NB,),
    in_specs=[
        pl.BlockSpec((BT, HIDDEN), lambda j: (j, 0)),
        pl.BlockSpec(memory_space=pl.ANY),
        pl.BlockSpec(memory_space=pl.ANY),
    ],
    out_specs=pl.BlockSpec(memory_space=pl.ANY),
    out_shape=jax.ShapeDtypeStruct((BATCH, VOCAB), jnp.float32),
    scratch_shapes=[
        pltpu.VMEM((VOCAB, HIDDEN), jnp.bfloat16),
        pltpu.VMEM((1, VOCAB), jnp.float32),
        pltpu.VMEM((BT, VOCAB), jnp.float32),
        pltpu.VMEM((BT, VOCAB), jnp.float32),
        pltpu.SemaphoreType.DMA,
        pltpu.SemaphoreType.DMA,
        pltpu.SemaphoreType.DMA,
        pltpu.SemaphoreType.DMA,
    ],
    compiler_params=pltpu.CompilerParams(
        dimension_semantics=("arbitrary",),
        vmem_limit_bytes=60 * 1024 * 1024,
    ),
)


def kernel(x, embed, W1, b1, W2, b2):
    table = jnp.pad(embed, ((0, 0), (0, EPAD - EMBED)))
    idx = x.reshape(-1, CHUNK).astype(jnp.int32)
    e = _make_gather()(idx, table)
    e2 = e.reshape(BATCH, CTX * EPAD)
    w1p = jnp.pad(
        W1.reshape(HIDDEN, CTX, EMBED), ((0, 0), (0, 0), (0, EPAD - EMBED))
    ).reshape(HIDDEN, CTX * EPAD)
    h = _fc1(e2, w1p, b1.reshape(1, HIDDEN))
    return _fc2(h, W2.astype(jnp.bfloat16), b2.reshape(1, VOCAB))


# speedup vs baseline: 1.4752x; 1.4752x over previous
import functools
import math

import jax
import jax.numpy as jnp
from jax import lax
from jax.experimental import pallas as pl
from jax.experimental.pallas import tpu as pltpu
from jax.experimental.pallas import tpu_sc as plsc

VOCAB = 100000
CTX = 32
EMBED = 7
HIDDEN = 64
BATCH = 1024
EPAD = 8
CHUNK = 128
VT = 4096
NV = math.ceil(VOCAB / VT)


@functools.lru_cache(maxsize=None)
def _make_gather():
    nc, ns = 2, 16
    nw = nc * ns
    total = BATCH * CTX
    rows_per_w = total // nw
    n_chunks = rows_per_w // CHUNK
    mesh = plsc.VectorSubcoreMesh(
        core_axis_name="c", subcore_axis_name="s", num_cores=nc, num_subcores=ns
    )

    @functools.partial(
        pl.kernel,
        mesh=mesh,
        compiler_params=pltpu.CompilerParams(use_tc_tiling_on_sc=False),
        out_type=jax.ShapeDtypeStruct((total, EPAD), jnp.float32),
        scratch_types=[
            pltpu.VMEM((n_chunks, CHUNK), jnp.int32),
            pltpu.VMEM((rows_per_w, EPAD), jnp.float32),
            pltpu.SemaphoreType.DMA,
        ],
    )
    def gather_k(idx_hbm, table_hbm, out_hbm, idx_v, rows_v, sem):
        wid = lax.axis_index("s") * nc + lax.axis_index("c")
        pltpu.sync_copy(idx_hbm.at[pl.ds(wid * n_chunks, n_chunks)], idx_v)
        copies = [
            pltpu.async_copy(
                table_hbm.at[idx_v.at[j]],
                rows_v.at[pl.ds(j * CHUNK, CHUNK)],
                sem,
            )
            for j in range(n_chunks)
        ]
        for c in copies:
            c.wait()
        pltpu.sync_copy(rows_v, out_hbm.at[pl.ds(wid * rows_per_w, rows_per_w)])

    return gather_k


def _mlp_body(e_ref, w1_ref, b1_ref, w2_ref, b2_ref, out_ref, h_ref):
    @pl.when(pl.program_id(0) == 0)
    def _():
        h = lax.dot_general(
            e_ref[...], w1_ref[...], (((1,), (1,)), ((), ())),
            preferred_element_type=jnp.float32,
        )
        h_ref[...] = jnp.maximum(h + b1_ref[...], 0.0)

    out_ref[...] = (
        lax.dot_general(
            h_ref[...], w2_ref[...], (((1,), (1,)), ((), ())),
            preferred_element_type=jnp.float32,
        )
        + b2_ref[...]
    )


_mlp = pl.pallas_call(
    _mlp_body,
    grid=(NV,),
    in_specs=[
        pl.BlockSpec((BATCH, CTX * EPAD), lambda i: (0, 0)),
        pl.BlockSpec((HIDDEN, CTX * EPAD), lambda i: (0, 0)),
        pl.BlockSpec((1, HIDDEN), lambda i: (0, 0)),
        pl.BlockSpec((VT, HIDDEN), lambda i: (i, 0)),
        pl.BlockSpec((1, VT), lambda i: (0, i)),
    ],
    out_specs=pl.BlockSpec((BATCH, VT), lambda i: (0, i)),
    out_shape=jax.ShapeDtypeStruct((BATCH, VOCAB), jnp.float32),
    scratch_shapes=[pltpu.VMEM((BATCH, HIDDEN), jnp.float32)],
    compiler_params=pltpu.CompilerParams(
        dimension_semantics=("arbitrary",),
    ),
)


def kernel(x, embed, W1, b1, W2, b2):
    table = jnp.pad(embed, ((0, 0), (0, EPAD - EMBED)))
    idx = x.reshape(-1, CHUNK).astype(jnp.int32)
    e = _make_gather()(idx, table)
    e2 = e.reshape(BATCH, CTX * EPAD)
    w1p = jnp.pad(
        W1.reshape(HIDDEN, CTX, EMBED), ((0, 0), (0, 0), (0, EPAD - EMBED))
    ).reshape(HIDDEN, CTX * EPAD)
    return _mlp(e2, w1p, b1.reshape(1, HIDDEN), W2, b2.reshape(1, VOCAB))
